# baseline (speedup 1.0000x reference)
import functools

import jax
import jax.numpy as jnp
from jax import lax
from jax.experimental import pallas as pl
from jax.experimental.pallas import tpu as pltpu
from jax.experimental.pallas import tpu_sc as plsc

N = 10000
E = 320000
D = 128
G = 128

NC = 2
NS = 16
NW = NC * NS
EPW = E // NW
CH = 50
NB = 4
PH = 5
CPP = 40
NITER = CPP // NB
RPT = 624
RTAIL = N - NS * RPT

BN = 1000
GRID = N // BN

_SC_MESH = plsc.VectorSubcoreMesh(core_axis_name="c", subcore_axis_name="s")


NPR = 80


def _deg_body(ei5_hbm, zeros_hbm, rid_hbm, out_hbm, dloc, degloc, rid, acc):
    c = lax.axis_index("c")
    s = lax.axis_index("s")
    wid = c * NS + s
    pltpu.sync_copy(zeros_hbm.at[pl.ds(0, NPR)], degloc)
    pltpu.sync_copy(rid_hbm, rid)

    @pl.when(s == 0)
    def _():
        pltpu.sync_copy(zeros_hbm.at[pl.ds(0, NPR)], acc)

    one16 = jnp.ones((16,), jnp.float32)
    tailmask = lax.broadcasted_iota(jnp.int32, (16,), 0) >= 8

    for p in range(PH):
        pltpu.sync_copy(ei5_hbm.at[1, wid, p], dloc)

        def sbody(cc, _):
            i0 = dloc[cc, pl.ds(0, 16)]
            plsc.addupdate_scatter(degloc, [i0 >> 7, i0 & 127], one16)
            i1 = dloc[cc, pl.ds(16, 16)]
            plsc.addupdate_scatter(degloc, [i1 >> 7, i1 & 127], one16)
            i2 = dloc[cc, pl.ds(24, 16)]
            plsc.addupdate_scatter(degloc, [i2 >> 7, i2 & 127], one16,
                                   mask=tailmask)
            return 0

        lax.fori_loop(0, CPP, sbody, 0)

    plsc.subcore_barrier()
    pltpu.sync_copy(degloc, acc.at[rid], add=True)
    plsc.subcore_barrier()

    @pl.when(s == 0)
    def _():
        pltpu.sync_copy(acc, out_hbm.at[c])


_deg_call = functools.partial(
    pl.kernel,
    out_type=jax.ShapeDtypeStruct((NC, NPR, 128), jnp.float32),
    mesh=_SC_MESH,
    compiler_params=pltpu.CompilerParams(needs_layout_passes=False),
    scratch_types=[
        pltpu.VMEM((CPP, CH), jnp.int32),
        pltpu.VMEM((NPR, 128), jnp.float32),
        pltpu.VMEM((NPR,), jnp.int32),
        pltpu.VMEM_SHARED((NPR, 128), jnp.float32),
    ],
)(_deg_body)


def _edge_body(ei5_hbm, hp_hbm, zeros_hbm, out_hbm, acc, sloc, dloc, *scr):
    c = lax.axis_index("c")
    s = lax.axis_index("s")
    wid = c * NS + s
    pltpu.sync_copy(zeros_hbm.at[pl.ds(s * RPT, RPT)],
                    acc.at[pl.ds(s * RPT, RPT)])

    @pl.when(s == 0)
    def _():
        pltpu.sync_copy(zeros_hbm.at[pl.ds(NS * RPT, RTAIL)],
                        acc.at[pl.ds(NS * RPT, RTAIL)])

    plsc.subcore_barrier()

    rows = list(scr[0:NB])
    isem = scr[NB]
    gsems = list(scr[NB + 1:2 * NB + 1])
    ssems = list(scr[2 * NB + 1:3 * NB + 1])

    for p in range(PH):
        di1 = pltpu.async_copy(ei5_hbm.at[0, wid, p], sloc, isem)
        di2 = pltpu.async_copy(ei5_hbm.at[1, wid, p], dloc, isem)
        di1.wait()
        di2.wait()
        for k in range(NB):
            pltpu.async_copy(hp_hbm.at[sloc.at[k]], rows[k], gsems[k])

        def body(j, _):
            base = j * NB
            for k in range(NB):
                ch = base + k
                pltpu.make_async_copy(hp_hbm.at[sloc.at[ch]], rows[k],
                                      gsems[k]).wait()
                pltpu.async_copy(rows[k], acc.at[dloc.at[ch]], ssems[k],
                                 add=True).wait()

                @pl.when(j < NITER - 1)
                def _():
                    pltpu.async_copy(hp_hbm.at[sloc.at[ch + NB]], rows[k],
                                     gsems[k])

            return 0

        lax.fori_loop(0, NITER, body, 0)

    plsc.subcore_barrier()
    pltpu.sync_copy(acc.at[pl.ds(s * RPT, RPT)],
                    out_hbm.at[pl.ds(c * N + s * RPT, RPT)])

    @pl.when(s == 0)
    def _():
        pltpu.sync_copy(acc.at[pl.ds(NS * RPT, RTAIL)],
                        out_hbm.at[pl.ds(c * N + NS * RPT, RTAIL)])


_edge_call = functools.partial(
    pl.kernel,
    out_type=jax.ShapeDtypeStruct((NC * N, D), jnp.float32),
    mesh=_SC_MESH,
    scratch_types=(
        [
            pltpu.VMEM_SHARED((N, D), jnp.float32),
            pltpu.VMEM((CPP, CH), jnp.int32),
            pltpu.VMEM((CPP, CH), jnp.int32),
        ]
        + [pltpu.VMEM((CH, D), jnp.float32)] * NB
        + [pltpu.SemaphoreType.DMA] * (2 * NB + 1)
    ),
)(_edge_body)


def _atoms_body(x_ref, wexp_ref, bexp_ref, w0_ref, atoms_ref, u0_ref):
    atoms = jnp.log(x_ref[...] + 1.0) @ wexp_ref[...] + bexp_ref[...]
    atoms_ref[...] = atoms
    u0_ref[...] = atoms @ w0_ref[...]


def _mkhp_body(u0_ref, deg_ref, dis_ref, hp_ref):
    dis = lax.rsqrt(deg_ref[...] + 1.0)
    dis_ref[...] = dis
    hp_ref[...] = u0_ref[...] * dis


def _finalize(p0, p1, hp, dis, b, g, be, atoms):
    agg = (p0 + p1 + hp) * dis + b
    mean = jnp.mean(agg, axis=-1, keepdims=True)
    var = jnp.mean((agg - mean) ** 2, axis=-1, keepdims=True)
    h = (agg - mean) * lax.rsqrt(var + 1e-5) * g + be
    h = 0.5 * h * (1.0 + lax.erf(h * 0.7071067811865475))
    return atoms + h


def _layer_body(p0_ref, p1_ref, hp_ref, dis_ref, b_ref, g_ref, be_ref,
                atoms_ref, wn_ref, atomsn_ref, hpn_ref):
    dis = dis_ref[...]
    atoms_n = _finalize(p0_ref[...], p1_ref[...], hp_ref[...], dis,
                        b_ref[...], g_ref[...], be_ref[...], atoms_ref[...])
    atomsn_ref[...] = atoms_n
    hpn_ref[...] = (atoms_n @ wn_ref[...]) * dis


def _final_body(p0_ref, p1_ref, hp_ref, dis_ref, b_ref, g_ref, be_ref,
                atoms_ref, batch_ref, out_ref):
    atoms_n = _finalize(p0_ref[...], p1_ref[...], hp_ref[...], dis_ref[...],
                        b_ref[...], g_ref[...], be_ref[...], atoms_ref[...])
    oh = (batch_ref[...] == lax.broadcasted_iota(jnp.int32, (BN, G), 1))
    contrib = lax.dot_general(oh.astype(jnp.float32), atoms_n,
                              (((0,), (0,)), ((), ())),
                              preferred_element_type=jnp.float32)

    @pl.when(pl.program_id(0) == 0)
    def _():
        out_ref[...] = jnp.zeros_like(out_ref)

    out_ref[...] += contrib


_ROW = pl.BlockSpec((BN, D), lambda i: (i, 0))
_ROW1 = pl.BlockSpec((BN, 1), lambda i: (i, 0))
_FULL_W = pl.BlockSpec((D, D), lambda i: (0, 0))
_FULL_V = pl.BlockSpec((D,), lambda i: (0,))
_P0 = pl.BlockSpec((BN, D), lambda i: (i, 0))
_P1 = pl.BlockSpec((BN, D), lambda i: (i + GRID, 0))

_atoms_call = pl.pallas_call(
    _atoms_body,
    grid=(GRID,),
    in_specs=[
        pl.BlockSpec((BN, 8), lambda i: (i, 0)),
        pl.BlockSpec((8, D), lambda i: (0, 0)),
        _FULL_V,
        _FULL_W,
    ],
    out_specs=[_ROW, _ROW],
    out_shape=[
        jax.ShapeDtypeStruct((N, D), jnp.float32),
        jax.ShapeDtypeStruct((N, D), jnp.float32),
    ],
)

_mkhp_call = pl.pallas_call(
    _mkhp_body,
    grid=(GRID,),
    in_specs=[_ROW, _ROW1],
    out_specs=[_ROW1, _ROW],
    out_shape=[
        jax.ShapeDtypeStruct((N, 1), jnp.float32),
        jax.ShapeDtypeStruct((N, D), jnp.float32),
    ],
)

_layer_call = pl.pallas_call(
    _layer_body,
    grid=(GRID,),
    in_specs=[_P0, _P1, _ROW, _ROW1, _FULL_V, _FULL_V, _FULL_V, _ROW, _FULL_W],
    out_specs=[_ROW, _ROW],
    out_shape=[
        jax.ShapeDtypeStruct((N, D), jnp.float32),
        jax.ShapeDtypeStruct((N, D), jnp.float32),
    ],
)

_final_call = pl.pallas_call(
    _final_body,
    grid=(GRID,),
    in_specs=[_P0, _P1, _ROW, _ROW1, _FULL_V, _FULL_V, _FULL_V, _ROW,
              pl.BlockSpec((BN, 1), lambda i: (i, 0))],
    out_specs=pl.BlockSpec((G, D), lambda i: (0, 0)),
    out_shape=jax.ShapeDtypeStruct((G, D), jnp.float32),
)


def kernel(x, edge_index, batch, Wexp, bexp,
           W0, b0, g0, be0, W1, b1, g1, be1, W2, b2, g2, be2):
    ei5 = edge_index.reshape(2, NW, PH, CPP, CH)
    zeros2d = jnp.zeros((N, D), jnp.float32)
    rid = jnp.arange(NPR, dtype=jnp.int32)

    degp = _deg_call(ei5, zeros2d, rid)
    atoms, u0 = _atoms_call(x, Wexp, bexp, W0)
    deg1 = (degp[0] + degp[1]).reshape(NPR * 128)[:N].reshape(N, 1)
    dis, hp = _mkhp_call(u0, deg1)

    params = [(b0, g0, be0, W1), (b1, g1, be1, W2), (b2, g2, be2, None)]
    for b, g, be, wn in params:
        part = _edge_call(ei5, hp, zeros2d)
        if wn is None:
            return _final_call(part, part, hp, dis, b, g, be, atoms,
                               batch.reshape(N, 1))
        atoms, hp = _layer_call(part, part, hp, dis, b, g, be, atoms, wn)


# speedup vs baseline: 1.0393x; 1.0393x over previous
import functools

import jax
import jax.numpy as jnp
from jax import lax
from jax.experimental import pallas as pl
from jax.experimental.pallas import tpu as pltpu
from jax.experimental.pallas import tpu_sc as plsc

N = 10000
E = 320000
D = 128
G = 128

NC = 2
NS = 16
NW = NC * NS
EPW = E // NW
CH = 40
NB = 5
PH = 10
CPP = 25
NITER = CPP // NB
RPT = 624
RTAIL = N - NS * RPT

BN = 1000
GRID = N // BN

_SC_MESH = plsc.VectorSubcoreMesh(core_axis_name="c", subcore_axis_name="s")


NPR = 80


def _deg_body(ei5_hbm, zeros_hbm, rid_hbm, out_hbm, dloc, degloc, rid, acc):
    c = lax.axis_index("c")
    s = lax.axis_index("s")
    wid = c * NS + s
    pltpu.sync_copy(zeros_hbm.at[pl.ds(0, NPR)], degloc)
    pltpu.sync_copy(rid_hbm, rid)

    @pl.when(s == 0)
    def _():
        pltpu.sync_copy(zeros_hbm.at[pl.ds(0, NPR)], acc)

    one16 = jnp.ones((16,), jnp.float32)
    tailmask = lax.broadcasted_iota(jnp.int32, (16,), 0) >= 8

    for p in range(PH):
        pltpu.sync_copy(ei5_hbm.at[1, wid, p], dloc)

        def sbody(cc, _):
            i0 = dloc[cc, pl.ds(0, 16)]
            plsc.addupdate_scatter(degloc, [i0 >> 7, i0 & 127], one16)
            i1 = dloc[cc, pl.ds(16, 16)]
            plsc.addupdate_scatter(degloc, [i1 >> 7, i1 & 127], one16)
            i2 = dloc[cc, pl.ds(24, 16)]
            plsc.addupdate_scatter(degloc, [i2 >> 7, i2 & 127], one16,
                                   mask=tailmask)
            return 0

        lax.fori_loop(0, CPP, sbody, 0)

    plsc.subcore_barrier()
    pltpu.sync_copy(degloc, acc.at[rid], add=True)
    plsc.subcore_barrier()

    @pl.when(s == 0)
    def _():
        pltpu.sync_copy(acc, out_hbm.at[c])


_deg_call = functools.partial(
    pl.kernel,
    out_type=jax.ShapeDtypeStruct((NC, NPR, 128), jnp.float32),
    mesh=_SC_MESH,
    compiler_params=pltpu.CompilerParams(needs_layout_passes=False),
    scratch_types=[
        pltpu.VMEM((CPP, CH), jnp.int32),
        pltpu.VMEM((NPR, 128), jnp.float32),
        pltpu.VMEM((NPR,), jnp.int32),
        pltpu.VMEM_SHARED((NPR, 128), jnp.float32),
    ],
)(_deg_body)


def _edge_body(ei5_hbm, hp_hbm, zeros_hbm, out_hbm, acc, sloc, dloc, *scr):
    c = lax.axis_index("c")
    s = lax.axis_index("s")
    wid = c * NS + s
    pltpu.sync_copy(zeros_hbm.at[pl.ds(s * RPT, RPT)],
                    acc.at[pl.ds(s * RPT, RPT)])

    @pl.when(s == 0)
    def _():
        pltpu.sync_copy(zeros_hbm.at[pl.ds(NS * RPT, RTAIL)],
                        acc.at[pl.ds(NS * RPT, RTAIL)])

    plsc.subcore_barrier()

    sloc2, dloc2 = scr[0], scr[1]
    rows = list(scr[2:NB + 2])
    isem = scr[NB + 2]
    gsems = list(scr[NB + 3:2 * NB + 3])
    ssems = list(scr[2 * NB + 3:3 * NB + 3])
    slocs = [sloc, sloc2]
    dlocs = [dloc, dloc2]

    di1 = pltpu.async_copy(ei5_hbm.at[0, wid, 0], sloc, isem)
    di2 = pltpu.async_copy(ei5_hbm.at[1, wid, 0], dloc, isem)
    di1.wait()
    di2.wait()
    for k in range(NB):
        pltpu.async_copy(hp_hbm.at[sloc.at[k]], rows[k], gsems[k])

    for p in range(PH):
        cs, cd = slocs[p % 2], dlocs[p % 2]
        ns, nd = slocs[(p + 1) % 2], dlocs[(p + 1) % 2]
        if p + 1 < PH:
            dn1 = pltpu.async_copy(ei5_hbm.at[0, wid, p + 1], ns, isem)
            dn2 = pltpu.async_copy(ei5_hbm.at[1, wid, p + 1], nd, isem)

        def body(j, _):
            base = j * NB
            for k in range(NB):
                ch = base + k
                pltpu.make_async_copy(hp_hbm.at[cs.at[ch]], rows[k],
                                      gsems[k]).wait()
                pltpu.async_copy(rows[k], acc.at[cd.at[ch]], ssems[k],
                                 add=True).wait()

                @pl.when(j < NITER - 1)
                def _():
                    pltpu.async_copy(hp_hbm.at[cs.at[ch + NB]], rows[k],
                                     gsems[k])

            return 0

        lax.fori_loop(0, NITER - 1, body, 0)
        base = (NITER - 1) * NB
        if p + 1 < PH:
            dn1.wait()
            dn2.wait()
        for k in range(NB):
            ch = base + k
            pltpu.make_async_copy(hp_hbm.at[cs.at[ch]], rows[k],
                                  gsems[k]).wait()
            pltpu.async_copy(rows[k], acc.at[cd.at[ch]], ssems[k],
                             add=True).wait()
            if p + 1 < PH:
                pltpu.async_copy(hp_hbm.at[ns.at[k]], rows[k], gsems[k])

    plsc.subcore_barrier()
    pltpu.sync_copy(acc.at[pl.ds(s * RPT, RPT)],
                    out_hbm.at[pl.ds(c * N + s * RPT, RPT)])

    @pl.when(s == 0)
    def _():
        pltpu.sync_copy(acc.at[pl.ds(NS * RPT, RTAIL)],
                        out_hbm.at[pl.ds(c * N + NS * RPT, RTAIL)])


_edge_call = functools.partial(
    pl.kernel,
    out_type=jax.ShapeDtypeStruct((NC * N, D), jnp.float32),
    mesh=_SC_MESH,
    scratch_types=(
        [
            pltpu.VMEM_SHARED((N, D), jnp.float32),
            pltpu.VMEM((CPP, CH), jnp.int32),
            pltpu.VMEM((CPP, CH), jnp.int32),
            pltpu.VMEM((CPP, CH), jnp.int32),
            pltpu.VMEM((CPP, CH), jnp.int32),
        ]
        + [pltpu.VMEM((CH, D), jnp.float32)] * NB
        + [pltpu.SemaphoreType.DMA] * (2 * NB + 1)
    ),
)(_edge_body)


def _atoms_body(x_ref, wexp_ref, bexp_ref, w0_ref, atoms_ref, u0_ref):
    atoms = jnp.log(x_ref[...] + 1.0) @ wexp_ref[...] + bexp_ref[...]
    atoms_ref[...] = atoms
    u0_ref[...] = atoms @ w0_ref[...]


def _mkhp_body(u0_ref, deg_ref, dis_ref, hp_ref):
    dis = lax.rsqrt(deg_ref[...] + 1.0)
    dis_ref[...] = dis
    hp_ref[...] = u0_ref[...] * dis


def _finalize(p0, p1, hp, dis, b, g, be, atoms):
    agg = (p0 + p1 + hp) * dis + b
    mean = jnp.mean(agg, axis=-1, keepdims=True)
    var = jnp.mean((agg - mean) ** 2, axis=-1, keepdims=True)
    h = (agg - mean) * lax.rsqrt(var + 1e-5) * g + be
    h = 0.5 * h * (1.0 + lax.erf(h * 0.7071067811865475))
    return atoms + h


def _layer_body(p0_ref, p1_ref, hp_ref, dis_ref, b_ref, g_ref, be_ref,
                atoms_ref, wn_ref, atomsn_ref, hpn_ref):
    dis = dis_ref[...]
    atoms_n = _finalize(p0_ref[...], p1_ref[...], hp_ref[...], dis,
                        b_ref[...], g_ref[...], be_ref[...], atoms_ref[...])
    atomsn_ref[...] = atoms_n
    hpn_ref[...] = (atoms_n @ wn_ref[...]) * dis


def _final_body(p0_ref, p1_ref, hp_ref, dis_ref, b_ref, g_ref, be_ref,
                atoms_ref, batch_ref, out_ref):
    atoms_n = _finalize(p0_ref[...], p1_ref[...], hp_ref[...], dis_ref[...],
                        b_ref[...], g_ref[...], be_ref[...], atoms_ref[...])
    oh = (batch_ref[...] == lax.broadcasted_iota(jnp.int32, (BN, G), 1))
    contrib = lax.dot_general(oh.astype(jnp.float32), atoms_n,
                              (((0,), (0,)), ((), ())),
                              preferred_element_type=jnp.float32)

    @pl.when(pl.program_id(0) == 0)
    def _():
        out_ref[...] = jnp.zeros_like(out_ref)

    out_ref[...] += contrib


_ROW = pl.BlockSpec((BN, D), lambda i: (i, 0))
_ROW1 = pl.BlockSpec((BN, 1), lambda i: (i, 0))
_FULL_W = pl.BlockSpec((D, D), lambda i: (0, 0))
_FULL_V = pl.BlockSpec((D,), lambda i: (0,))
_P0 = pl.BlockSpec((BN, D), lambda i: (i, 0))
_P1 = pl.BlockSpec((BN, D), lambda i: (i + GRID, 0))

_atoms_call = pl.pallas_call(
    _atoms_body,
    grid=(GRID,),
    in_specs=[
        pl.BlockSpec((BN, 8), lambda i: (i, 0)),
        pl.BlockSpec((8, D), lambda i: (0, 0)),
        _FULL_V,
        _FULL_W,
    ],
    out_specs=[_ROW, _ROW],
    out_shape=[
        jax.ShapeDtypeStruct((N, D), jnp.float32),
        jax.ShapeDtypeStruct((N, D), jnp.float32),
    ],
)

_mkhp_call = pl.pallas_call(
    _mkhp_body,
    grid=(GRID,),
    in_specs=[_ROW, _ROW1],
    out_specs=[_ROW1, _ROW],
    out_shape=[
        jax.ShapeDtypeStruct((N, 1), jnp.float32),
        jax.ShapeDtypeStruct((N, D), jnp.float32),
    ],
)

_layer_call = pl.pallas_call(
    _layer_body,
    grid=(GRID,),
    in_specs=[_P0, _P1, _ROW, _ROW1, _FULL_V, _FULL_V, _FULL_V, _ROW, _FULL_W],
    out_specs=[_ROW, _ROW],
    out_shape=[
        jax.ShapeDtypeStruct((N, D), jnp.float32),
        jax.ShapeDtypeStruct((N, D), jnp.float32),
    ],
)

_final_call = pl.pallas_call(
    _final_body,
    grid=(GRID,),
    in_specs=[_P0, _P1, _ROW, _ROW1, _FULL_V, _FULL_V, _FULL_V, _ROW,
              pl.BlockSpec((BN, 1), lambda i: (i, 0))],
    out_specs=pl.BlockSpec((G, D), lambda i: (0, 0)),
    out_shape=jax.ShapeDtypeStruct((G, D), jnp.float32),
)


def kernel(x, edge_index, batch, Wexp, bexp,
           W0, b0, g0, be0, W1, b1, g1, be1, W2, b2, g2, be2):
    ei5 = edge_index.reshape(2, NW, PH, CPP, CH)
    zeros2d = jnp.zeros((N, D), jnp.float32)
    rid = jnp.arange(NPR, dtype=jnp.int32)

    degp = _deg_call(ei5, zeros2d, rid)
    atoms, u0 = _atoms_call(x, Wexp, bexp, W0)
    deg1 = (degp[0] + degp[1]).reshape(NPR * 128)[:N].reshape(N, 1)
    dis, hp = _mkhp_call(u0, deg1)

    params = [(b0, g0, be0, W1), (b1, g1, be1, W2), (b2, g2, be2, None)]
    for b, g, be, wn in params:
        part = _edge_call(ei5, hp, zeros2d)
        if wn is None:
            return _final_call(part, part, hp, dis, b, g, be, atoms,
                               batch.reshape(N, 1))
        atoms, hp = _layer_call(part, part, hp, dis, b, g, be, atoms, wn)


# speedup vs baseline: 1.0407x; 1.0013x over previous
import functools

import jax
import jax.numpy as jnp
from jax import lax
from jax.experimental import pallas as pl
from jax.experimental.pallas import tpu as pltpu
from jax.experimental.pallas import tpu_sc as plsc

N = 10000
E = 320000
D = 128
G = 128

NC = 2
NS = 16
NW = NC * NS
EPW = E // NW
CH = 40
NB = 5
PH = 10
CPP = 25
NITER = CPP // NB
RPT = 624
RTAIL = N - NS * RPT

BN = 1000
GRID = N // BN

_SC_MESH = plsc.VectorSubcoreMesh(core_axis_name="c", subcore_axis_name="s")


NPR = 80


def _deg_body(dstT_hbm, zeros_hbm, rid_hbm, out_hbm, dloc, degloc, rid, acc):
    c = lax.axis_index("c")
    s = lax.axis_index("s")
    wid = c * NS + s
    pltpu.sync_copy(zeros_hbm.at[pl.ds(0, NPR)], degloc)
    pltpu.sync_copy(rid_hbm, rid)

    @pl.when(s == 0)
    def _():
        pltpu.sync_copy(zeros_hbm.at[pl.ds(0, NPR)], acc)

    one16 = jnp.ones((16,), jnp.float32)
    tailmask = lax.broadcasted_iota(jnp.int32, (16,), 0) >= 8

    for p in range(PH):
        pltpu.sync_copy(dstT_hbm.at[wid, p], dloc)

        def sbody(cc, _):
            i0 = dloc[cc, pl.ds(0, 16)]
            plsc.addupdate_scatter(degloc, [i0 >> 7, i0 & 127], one16)
            i1 = dloc[cc, pl.ds(16, 16)]
            plsc.addupdate_scatter(degloc, [i1 >> 7, i1 & 127], one16)
            i2 = dloc[cc, pl.ds(24, 16)]
            plsc.addupdate_scatter(degloc, [i2 >> 7, i2 & 127], one16,
                                   mask=tailmask)
            return 0

        lax.fori_loop(0, CPP, sbody, 0)

    plsc.subcore_barrier()
    pltpu.sync_copy(degloc, acc.at[rid], add=True)
    plsc.subcore_barrier()

    @pl.when(s == 0)
    def _():
        pltpu.sync_copy(acc, out_hbm.at[c])


_deg_call = functools.partial(
    pl.kernel,
    out_type=jax.ShapeDtypeStruct((NC, NPR, 128), jnp.float32),
    mesh=_SC_MESH,
    compiler_params=pltpu.CompilerParams(needs_layout_passes=False),
    scratch_types=[
        pltpu.VMEM((CPP, CH), jnp.int32),
        pltpu.VMEM((NPR, 128), jnp.float32),
        pltpu.VMEM((NPR,), jnp.int32),
        pltpu.VMEM_SHARED((NPR, 128), jnp.float32),
    ],
)(_deg_body)


def _edge_body(srcT_hbm, dstT_hbm, hp_hbm, zeros_hbm, out_hbm,
               acc, sloc, dloc, *scr):
    c = lax.axis_index("c")
    s = lax.axis_index("s")
    wid = c * NS + s
    pltpu.sync_copy(zeros_hbm.at[pl.ds(s * RPT, RPT)],
                    acc.at[pl.ds(s * RPT, RPT)])

    @pl.when(s == 0)
    def _():
        pltpu.sync_copy(zeros_hbm.at[pl.ds(NS * RPT, RTAIL)],
                        acc.at[pl.ds(NS * RPT, RTAIL)])

    plsc.subcore_barrier()

    sloc2, dloc2 = scr[0], scr[1]
    rows = list(scr[2:NB + 2])
    isem = scr[NB + 2]
    gsems = list(scr[NB + 3:2 * NB + 3])
    ssems = list(scr[2 * NB + 3:3 * NB + 3])
    slocs = [sloc, sloc2]
    dlocs = [dloc, dloc2]

    di1 = pltpu.async_copy(srcT_hbm.at[wid, 0], sloc, isem)
    di2 = pltpu.async_copy(dstT_hbm.at[wid, 0], dloc, isem)
    di1.wait()
    di2.wait()
    for k in range(NB):
        pltpu.async_copy(hp_hbm.at[sloc.at[pl.ds(k * CH, CH)]], rows[k],
                         gsems[k])

    for p in range(PH):
        cs, cd = slocs[p % 2], dlocs[p % 2]
        ns, nd = slocs[(p + 1) % 2], dlocs[(p + 1) % 2]
        if p + 1 < PH:
            dn1 = pltpu.async_copy(srcT_hbm.at[wid, p + 1], ns, isem)
            dn2 = pltpu.async_copy(dstT_hbm.at[wid, p + 1], nd, isem)

        def body(j, _):
            base = j * NB
            for k in range(NB):
                ch = base + k
                pltpu.make_async_copy(hp_hbm.at[cs.at[pl.ds(ch * CH, CH)]],
                                      rows[k], gsems[k]).wait()
                pltpu.async_copy(rows[k], acc.at[cd.at[ch]], ssems[k],
                                 add=True).wait()

                @pl.when(j < NITER - 1)
                def _():
                    pltpu.async_copy(
                        hp_hbm.at[cs.at[pl.ds((ch + NB) * CH, CH)]],
                        rows[k], gsems[k])

            return 0

        lax.fori_loop(0, NITER - 1, body, 0)
        base = (NITER - 1) * NB
        if p + 1 < PH:
            dn1.wait()
            dn2.wait()
        for k in range(NB):
            ch = base + k
            pltpu.make_async_copy(hp_hbm.at[cs.at[pl.ds(ch * CH, CH)]],
                                  rows[k], gsems[k]).wait()
            pltpu.async_copy(rows[k], acc.at[cd.at[ch]], ssems[k],
                             add=True).wait()
            if p + 1 < PH:
                pltpu.async_copy(hp_hbm.at[ns.at[pl.ds(k * CH, CH)]],
                                 rows[k], gsems[k])

    plsc.subcore_barrier()
    pltpu.sync_copy(acc.at[pl.ds(s * RPT, RPT)],
                    out_hbm.at[pl.ds(c * N + s * RPT, RPT)])

    @pl.when(s == 0)
    def _():
        pltpu.sync_copy(acc.at[pl.ds(NS * RPT, RTAIL)],
                        out_hbm.at[pl.ds(c * N + NS * RPT, RTAIL)])


_edge_call = functools.partial(
    pl.kernel,
    out_type=jax.ShapeDtypeStruct((NC * N, D), jnp.float32),
    mesh=_SC_MESH,
    scratch_types=(
        [
            pltpu.VMEM_SHARED((N, D), jnp.float32),
            pltpu.VMEM((CPP * CH,), jnp.int32),
            pltpu.VMEM((CPP, CH), jnp.int32),
            pltpu.VMEM((CPP * CH,), jnp.int32),
            pltpu.VMEM((CPP, CH), jnp.int32),
        ]
        + [pltpu.VMEM((CH, D), jnp.float32)] * NB
        + [pltpu.SemaphoreType.DMA] * (2 * NB + 1)
    ),
)(_edge_body)


def _atoms_body(x_ref, wexp_ref, bexp_ref, w0_ref, atoms_ref, u0_ref):
    atoms = jnp.log(x_ref[...] + 1.0) @ wexp_ref[...] + bexp_ref[...]
    atoms_ref[...] = atoms
    u0_ref[...] = atoms @ w0_ref[...]


def _mkhp_body(u0_ref, deg_ref, dis_ref, hp_ref):
    dis = lax.rsqrt(deg_ref[...] + 1.0)
    dis_ref[...] = dis
    hp_ref[...] = u0_ref[...] * dis


def _finalize(p0, p1, hp, dis, b, g, be, atoms):
    agg = (p0 + p1 + hp) * dis + b
    mean = jnp.mean(agg, axis=-1, keepdims=True)
    var = jnp.mean((agg - mean) ** 2, axis=-1, keepdims=True)
    h = (agg - mean) * lax.rsqrt(var + 1e-5) * g + be
    h = 0.5 * h * (1.0 + lax.erf(h * 0.7071067811865475))
    return atoms + h


def _layer_body(p0_ref, p1_ref, hp_ref, dis_ref, b_ref, g_ref, be_ref,
                atoms_ref, wn_ref, atomsn_ref, hpn_ref):
    dis = dis_ref[...]
    atoms_n = _finalize(p0_ref[...], p1_ref[...], hp_ref[...], dis,
                        b_ref[...], g_ref[...], be_ref[...], atoms_ref[...])
    atomsn_ref[...] = atoms_n
    hpn_ref[...] = (atoms_n @ wn_ref[...]) * dis


def _final_body(p0_ref, p1_ref, hp_ref, dis_ref, b_ref, g_ref, be_ref,
                atoms_ref, batch_ref, out_ref):
    atoms_n = _finalize(p0_ref[...], p1_ref[...], hp_ref[...], dis_ref[...],
                        b_ref[...], g_ref[...], be_ref[...], atoms_ref[...])
    oh = (batch_ref[...] == lax.broadcasted_iota(jnp.int32, (BN, G), 1))
    contrib = lax.dot_general(oh.astype(jnp.float32), atoms_n,
                              (((0,), (0,)), ((), ())),
                              preferred_element_type=jnp.float32)

    @pl.when(pl.program_id(0) == 0)
    def _():
        out_ref[...] = jnp.zeros_like(out_ref)

    out_ref[...] += contrib


_ROW = pl.BlockSpec((BN, D), lambda i: (i, 0))
_ROW1 = pl.BlockSpec((BN, 1), lambda i: (i, 0))
_FULL_W = pl.BlockSpec((D, D), lambda i: (0, 0))
_FULL_V = pl.BlockSpec((D,), lambda i: (0,))
_P0 = pl.BlockSpec((BN, D), lambda i: (i, 0))
_P1 = pl.BlockSpec((BN, D), lambda i: (i + GRID, 0))

_atoms_call = pl.pallas_call(
    _atoms_body,
    grid=(GRID,),
    in_specs=[
        pl.BlockSpec((BN, 8), lambda i: (i, 0)),
        pl.BlockSpec((8, D), lambda i: (0, 0)),
        _FULL_V,
        _FULL_W,
    ],
    out_specs=[_ROW, _ROW],
    out_shape=[
        jax.ShapeDtypeStruct((N, D), jnp.float32),
        jax.ShapeDtypeStruct((N, D), jnp.float32),
    ],
)

_mkhp_call = pl.pallas_call(
    _mkhp_body,
    grid=(GRID,),
    in_specs=[_ROW, _ROW1],
    out_specs=[_ROW1, _ROW],
    out_shape=[
        jax.ShapeDtypeStruct((N, 1), jnp.float32),
        jax.ShapeDtypeStruct((N, D), jnp.float32),
    ],
)

_layer_call = pl.pallas_call(
    _layer_body,
    grid=(GRID,),
    in_specs=[_P0, _P1, _ROW, _ROW1, _FULL_V, _FULL_V, _FULL_V, _ROW, _FULL_W],
    out_specs=[_ROW, _ROW],
    out_shape=[
        jax.ShapeDtypeStruct((N, D), jnp.float32),
        jax.ShapeDtypeStruct((N, D), jnp.float32),
    ],
)

_final_call = pl.pallas_call(
    _final_body,
    grid=(GRID,),
    in_specs=[_P0, _P1, _ROW, _ROW1, _FULL_V, _FULL_V, _FULL_V, _ROW,
              pl.BlockSpec((BN, 1), lambda i: (i, 0))],
    out_specs=pl.BlockSpec((G, D), lambda i: (0, 0)),
    out_shape=jax.ShapeDtypeStruct((G, D), jnp.float32),
)


def kernel(x, edge_index, batch, Wexp, bexp,
           W0, b0, g0, be0, W1, b1, g1, be1, W2, b2, g2, be2):
    srcT = edge_index[0].reshape(NW, PH, CPP * CH)
    dstT = edge_index[1].reshape(NW, PH, CPP, CH)
    zeros2d = jnp.zeros((N, D), jnp.float32)
    rid = jnp.arange(NPR, dtype=jnp.int32)

    degp = _deg_call(dstT, zeros2d, rid)
    atoms, u0 = _atoms_call(x, Wexp, bexp, W0)
    deg1 = (degp[0] + degp[1]).reshape(NPR * 128)[:N].reshape(N, 1)
    dis, hp = _mkhp_call(u0, deg1)

    params = [(b0, g0, be0, W1), (b1, g1, be1, W2), (b2, g2, be2, None)]
    for b, g, be, wn in params:
        part = _edge_call(srcT, dstT, hp, zeros2d)
        if wn is None:
            return _final_call(part, part, hp, dis, b, g, be, atoms,
                               batch.reshape(N, 1))
        atoms, hp = _layer_call(part, part, hp, dis, b, g, be, atoms, wn)
